# baseline (speedup 1.0000x reference)
import functools

import jax
import jax.numpy as jnp
from jax import lax
from jax.experimental import pallas as pl
from jax.experimental.pallas import tpu as pltpu
from jax.experimental.pallas import tpu_sc as plsc

NUM_EMB = 1000000
DIM = 32
NC = 2
NS = 16
NW = NC * NS
UNIT = 1024
TT = 25
BT = 128
N_UNITS = TT * BT
PER_W = N_UNITS // NW


def _make_lookup():
  mesh = plsc.VectorSubcoreMesh(core_axis_name="c", subcore_axis_name="s")

  @functools.partial(
      pl.kernel,
      mesh=mesh,
      out_type=jax.ShapeDtypeStruct((200, 4, BT, 8, 128), jnp.float32),
      compiler_params=pltpu.CompilerParams(
          use_tc_tiling_on_sc=False, needs_layout_passes=False),
      scratch_types=[
          pltpu.VMEM((2, UNIT), jnp.int32),
          pltpu.VMEM((2, UNIT, DIM), jnp.float32),
          pltpu.VMEM((2, 4, 4, 8, 129), jnp.float32),
          [pltpu.SemaphoreType.DMA] * 2,
          [pltpu.SemaphoreType.DMA] * 2,
          [pltpu.SemaphoreType.DMA] * 2,
      ],
  )
  def lookup(idx_hbm, table_hbm, z_hbm, idx_v, rows_v, zbuf, sidx, sgat, szout):
    wid = lax.axis_index("s") * NC + lax.axis_index("c")
    base = wid * PER_W

    def idx_copy(u, j):
      return pltpu.make_async_copy(
          idx_hbm.at[pl.ds(u * UNIT, UNIT)], idx_v.at[j], sidx[j])

    def gather_copy(j):
      return pltpu.make_async_copy(
          table_hbm.at[idx_v.at[j]], rows_v.at[j], sgat[j])

    def zout_copy(u, h):
      tt = u // BT
      bt = u % BT
      return pltpu.make_async_copy(
          zbuf.at[h, :, :, :, pl.ds(0, 128)],
          z_hbm.at[pl.ds(tt * 8 + h * 4, 4), :, bt], szout[h])

    dlo = lax.iota(jnp.int32, 16)
    dhi = dlo + 16

    def transpose_half(j, h):
      def body(tv, carry):
        si = tv // 128
        l = tv % 128
        tok = h * 512 + tv
        lv = jnp.full((16,), l, jnp.int32)
        a = rows_v[j, tok, pl.ds(0, 16)]
        b = rows_v[j, tok, pl.ds(16, 16)]
        zsub = zbuf.at[h, si]
        plsc.store_scatter(zsub, [dlo // 8, dlo % 8, lv], a)
        plsc.store_scatter(zsub, [dhi // 8, dhi % 8, lv], b)
        return carry

      lax.fori_loop(0, 512, body, 0, unroll=4)

    idx_copy(base, 0).start()
    idx_copy(base + 1, 1).start()
    idx_copy(base, 0).wait()
    gather_copy(0).start()

    def step(i, j):
      u = base + i
      j2 = 1 - j
      gather_copy(j).wait()
      def _next_gather():
        idx_copy(u + 1, j2).wait()
        gather_copy(j2).start()

      pl.when(i + 1 < PER_W)(_next_gather)
      pl.when(i + 2 < PER_W)(lambda: idx_copy(u + 2, j).start())
      for h in range(2):
        pl.when(i > 0)(lambda h=h: zout_copy(u - 1, h).wait())
        transpose_half(j, h)
        zout_copy(u, h).start()

    def pair(g, carry):
      step(g * 2, 0)
      step(g * 2 + 1, 1)
      return carry

    lax.fori_loop(0, PER_W // 2, pair, 0, unroll=False)
    zout_copy(base + PER_W - 1, 0).wait()
    zout_copy(base + PER_W - 1, 1).wait()

  return lookup


def kernel(token_ids, weight):
  tid_lin = (
      token_ids.T.reshape(TT, 8, BT, 128).transpose(0, 2, 1, 3).reshape(-1)
  ).astype(jnp.int32)
  z = _make_lookup()(tid_lin, weight)
  return z.transpose(2, 4, 0, 1, 3).reshape(16384, 200, DIM)


# speedup vs baseline: 1.0646x; 1.0646x over previous
import functools

import jax
import jax.numpy as jnp
from jax import lax
from jax.experimental import pallas as pl
from jax.experimental.pallas import tpu as pltpu
from jax.experimental.pallas import tpu_sc as plsc

NUM_EMB = 1000000
DIM = 32
NC = 2
NS = 16
NW = NC * NS
UNIT = 1024
TT = 25
BT = 128
N_UNITS = TT * BT
PER_W = N_UNITS // NW


def _make_lookup():
  mesh = plsc.VectorSubcoreMesh(core_axis_name="c", subcore_axis_name="s")

  @functools.partial(
      pl.kernel,
      mesh=mesh,
      out_type=jax.ShapeDtypeStruct((200, 4, BT, 8, 128), jnp.float32),
      compiler_params=pltpu.CompilerParams(
          use_tc_tiling_on_sc=False, needs_layout_passes=False),
      scratch_types=[
          pltpu.VMEM((2, UNIT), jnp.int32),
          pltpu.VMEM((2, UNIT, DIM), jnp.float32),
          pltpu.VMEM((2, 4, 4, 8, 129), jnp.float32),
          [pltpu.SemaphoreType.DMA] * 2,
          [pltpu.SemaphoreType.DMA] * 2,
          [pltpu.SemaphoreType.DMA] * 2,
      ],
  )
  def lookup(idx_hbm, table_hbm, z_hbm, idx_v, rows_v, zbuf, sidx, sgat, szout):
    wid = lax.axis_index("s") * NC + lax.axis_index("c")
    base = wid * PER_W

    def idx_copy(u, j):
      return pltpu.make_async_copy(
          idx_hbm.at[pl.ds(u * UNIT, UNIT)], idx_v.at[j], sidx[j])

    def gather_copy(j):
      return pltpu.make_async_copy(
          table_hbm.at[idx_v.at[j]], rows_v.at[j], sgat[j])

    def zout_copy(u, h):
      tt = u // BT
      bt = u % BT
      return pltpu.make_async_copy(
          zbuf.at[h, :, :, :, pl.ds(0, 128)],
          z_hbm.at[pl.ds(tt * 8 + h * 4, 4), :, bt], szout[h])

    dlo = lax.iota(jnp.int32, 16)
    dhi = dlo + 16

    def transpose_half(j, h):
      def body(tv, carry):
        si = tv // 128
        l = tv % 128
        tok = h * 512 + tv
        lv = jnp.full((16,), l, jnp.int32)
        a = rows_v[j, tok, pl.ds(0, 16)]
        b = rows_v[j, tok, pl.ds(16, 16)]
        zsub = zbuf.at[h, si]
        plsc.store_scatter(zsub, [dlo // 8, dlo % 8, lv], a)
        plsc.store_scatter(zsub, [dhi // 8, dhi % 8, lv], b)
        return carry

      lax.fori_loop(0, 512, body, 0, unroll=False)

    idx_copy(base, 0).start()
    idx_copy(base + 1, 1).start()
    idx_copy(base, 0).wait()
    gather_copy(0).start()

    def step(i, j):
      u = base + i
      j2 = 1 - j
      gather_copy(j).wait()
      def _next_gather():
        idx_copy(u + 1, j2).wait()
        gather_copy(j2).start()

      pl.when(i + 1 < PER_W)(_next_gather)
      pl.when(i + 2 < PER_W)(lambda: idx_copy(u + 2, j).start())
      for h in range(2):
        pl.when(i > 0)(lambda h=h: zout_copy(u - 1, h).wait())
        transpose_half(j, h)
        zout_copy(u, h).start()

    def pair(g, carry):
      step(g * 2, 0)
      step(g * 2 + 1, 1)
      return carry

    lax.fori_loop(0, PER_W // 2, pair, 0, unroll=False)
    zout_copy(base + PER_W - 1, 0).wait()
    zout_copy(base + PER_W - 1, 1).wait()

  return lookup


VT = 7813
GROUPS = 1953


def _make_weight_linearize():
  mesh = plsc.VectorSubcoreMesh(core_axis_name="c", subcore_axis_name="s")

  @functools.partial(
      pl.kernel,
      mesh=mesh,
      out_type=jax.ShapeDtypeStruct((NUM_EMB, DIM), jnp.float32),
      compiler_params=pltpu.CompilerParams(
          use_tc_tiling_on_sc=False, needs_layout_passes=False),
      scratch_types=[
          pltpu.VMEM((2, 4, 4, 8, 128), jnp.float32),
          pltpu.VMEM((2, 512, DIM + 1), jnp.float32),
          [pltpu.SemaphoreType.DMA] * 2,
          [pltpu.SemaphoreType.DMA] * 2,
      ],
  )
  def linearize(wv_hbm, o_hbm, tbuf, obuf, sin, sout):
    wid = lax.axis_index("s") * NC + lax.axis_index("c")
    lanes = lax.iota(jnp.int32, 16)

    def in_copy(g, j):
      return pltpu.make_async_copy(
          wv_hbm.at[:, pl.ds(g * 4, 4)], tbuf.at[j], sin[j])

    def out_copy(g, j):
      return pltpu.make_async_copy(
          obuf.at[j, :, pl.ds(0, DIM)],
          o_hbm.at[pl.ds(g * 512, 512)], sout[j])

    def transpose_group(j):
      def body(it, carry):
        vtl = it // DIM
        d = it % DIM
        dsplat = jnp.full((16,), d, jnp.int32)
        for v in range(8):
          vals = tbuf[j, d // 8, vtl, d % 8, pl.ds(v * 16, 16)]
          ridx = vtl * 128 + v * 16 + lanes
          plsc.store_scatter(obuf.at[j], [ridx, dsplat], vals)
        return carry

      lax.fori_loop(0, 4 * DIM, body, 0, unroll=False)

    def step(k, j):
      g = k * NW + wid

      def work():
        in_copy(g, j).wait()
        pl.when(k >= 2)(lambda: out_copy((k - 2) * NW + wid, j).wait())
        transpose_group(j)
        out_copy(g, j).start()
        g2 = g + 2 * NW
        pl.when(g2 < GROUPS)(lambda: in_copy(g2, j).start())

      pl.when(g < GROUPS)(work)

    pl.when(wid < GROUPS)(lambda: in_copy(wid, 0).start())
    pl.when(wid + NW < GROUPS)(lambda: in_copy(wid + NW, 1).start())

    def pair(kk, carry):
      step(kk * 2, 0)
      step(kk * 2 + 1, 1)
      return carry

    n_k = (GROUPS + NW - 1) // NW
    lax.fori_loop(0, (n_k + 1) // 2, pair, 0, unroll=False)
    for last_k, j in ((60, 0), (61, 1)):
      pl.when(last_k * NW + wid < GROUPS)(
          lambda j=j, last_k=last_k: out_copy(last_k * NW + wid, j).wait())

    def tail():
      vt = GROUPS * 4
      pltpu.sync_copy(wv_hbm.at[:, vt], tbuf.at[0, :, 0])

      def body(d, carry):
        dsplat = jnp.full((16,), d, jnp.int32)
        for v in range(4):
          vals = tbuf[0, d // 8, 0, d % 8, pl.ds(v * 16, 16)]
          ridx = v * 16 + lanes
          plsc.store_scatter(obuf.at[0], [ridx, dsplat], vals)
        return carry

      lax.fori_loop(0, DIM, body, 0, unroll=False)
      pltpu.sync_copy(
          obuf.at[0, pl.ds(0, 64), pl.ds(0, DIM)],
          o_hbm.at[pl.ds(999936, 64)])

    pl.when(wid == 0)(tail)

  return linearize


def kernel(token_ids, weight):
  tid_lin = (
      token_ids.T.reshape(TT, 8, BT, 128).transpose(0, 2, 1, 3).reshape(-1)
  ).astype(jnp.int32)
  wv = (
      jnp.pad(weight.T, ((0, 0), (0, 64)))
      .reshape(4, 8, VT, 128).transpose(0, 2, 1, 3)
  )
  wlin = _make_weight_linearize()(wv)
  z = _make_lookup()(tid_lin, wlin)
  return z.transpose(2, 4, 0, 1, 3).reshape(16384, 200, DIM)
